# Initial kernel scaffold; baseline (speedup 1.0000x reference)
import jax
import jax.numpy as jnp
from jax.experimental import pallas as pl


def kernel(cond_ids, emb_weight):
    raise NotImplementedError("write your pallas kernel here")


# speedup vs baseline: 1.4366x; 1.4366x over previous
import functools

import jax
import jax.numpy as jnp
from jax import lax
from jax.experimental import pallas as pl
from jax.experimental.pallas import tpu as pltpu
from jax.experimental.pallas import tpu_sc as plsc

EMB_DIM = 32

NC = 2
NS = 16
NW = NC * NS

CHUNK = 128


@functools.partial(jax.jit, static_argnums=(0,))
def _gather_rows(B, idx, table):
    b_per_w = B // NW
    n_chunks = b_per_w // CHUNK
    mesh = plsc.VectorSubcoreMesh(core_axis_name="c", subcore_axis_name="s")

    @functools.partial(
        pl.kernel,
        out_type=jax.ShapeDtypeStruct((B, EMB_DIM), jnp.float32),
        mesh=mesh,
        scratch_types=[
            pltpu.VMEM((n_chunks, CHUNK), jnp.int32),
            pltpu.VMEM((CHUNK, EMB_DIM), jnp.float32),
            pltpu.SemaphoreType.DMA,
        ],
        compiler_params=pltpu.CompilerParams(use_tc_tiling_on_sc=False),
    )
    def k(idx_hbm, table_hbm, out_hbm, idx_v, rows_v, sem):
        wid = lax.axis_index("s") * NC + lax.axis_index("c")
        base = wid * b_per_w
        pltpu.sync_copy(idx_hbm.at[wid], idx_v)

        def body(c, carry):
            pltpu.async_copy(table_hbm.at[idx_v.at[c]], rows_v, sem).wait()
            pltpu.sync_copy(rows_v, out_hbm.at[pl.ds(base + c * CHUNK, CHUNK)])
            return carry

        lax.fori_loop(0, n_chunks, body, 0)

    return k(idx, table)


def kernel(cond_ids, emb_weight):
    batch, nf = cond_ids.shape
    B = batch * nf
    idx = cond_ids.astype(jnp.int32).reshape(NW, B // NW // CHUNK, CHUNK)
    out = _gather_rows(B, idx, emb_weight)
    return out.reshape(batch, nf, EMB_DIM)


# speedup vs baseline: 1.5587x; 1.0850x over previous
import functools

import jax
import jax.numpy as jnp
from jax import lax
from jax.experimental import pallas as pl
from jax.experimental.pallas import tpu as pltpu
from jax.experimental.pallas import tpu_sc as plsc

EMB_DIM = 32

NC = 2
NS = 16
NW = NC * NS

CHUNK = 1024


@functools.partial(jax.jit, static_argnums=(0,))
def _gather_rows(B, idx, table):
    b_per_w = B // NW
    n_chunks = b_per_w // CHUNK
    mesh = plsc.VectorSubcoreMesh(core_axis_name="c", subcore_axis_name="s")

    @functools.partial(
        pl.kernel,
        out_type=jax.ShapeDtypeStruct((B, EMB_DIM), jnp.float32),
        mesh=mesh,
        scratch_types=[
            pltpu.VMEM((n_chunks, CHUNK), jnp.int32),
            pltpu.VMEM((CHUNK, EMB_DIM), jnp.float32),
            pltpu.SemaphoreType.DMA,
        ],
        compiler_params=pltpu.CompilerParams(use_tc_tiling_on_sc=False),
    )
    def k(idx_hbm, table_hbm, out_hbm, idx_v, rows_v, sem):
        wid = lax.axis_index("s") * NC + lax.axis_index("c")
        base = wid * b_per_w
        pltpu.sync_copy(idx_hbm.at[wid], idx_v)

        def body(c, carry):
            pltpu.async_copy(table_hbm.at[idx_v.at[c]], rows_v, sem).wait()
            pltpu.sync_copy(rows_v, out_hbm.at[pl.ds(base + c * CHUNK, CHUNK)])
            return carry

        lax.fori_loop(0, n_chunks, body, 0)

    return k(idx, table)


def kernel(cond_ids, emb_weight):
    batch, nf = cond_ids.shape
    B = batch * nf
    idx = cond_ids.astype(jnp.int32).reshape(NW, B // NW // CHUNK, CHUNK)
    out = _gather_rows(B, idx, emb_weight)
    return out.reshape(batch, nf, EMB_DIM)


# speedup vs baseline: 1.5767x; 1.0116x over previous
import functools

import jax
import jax.numpy as jnp
from jax import lax
from jax.experimental import pallas as pl
from jax.experimental.pallas import tpu as pltpu
from jax.experimental.pallas import tpu_sc as plsc

EMB_DIM = 32

NC = 2
NS = 16
NW = NC * NS

CHUNK = 1024


@functools.partial(jax.jit, static_argnums=(0,))
def _gather_rows(B, idx, table):
    b_per_w = B // NW
    n_chunks = b_per_w // CHUNK
    mesh = plsc.VectorSubcoreMesh(core_axis_name="c", subcore_axis_name="s")

    @functools.partial(
        pl.kernel,
        out_type=jax.ShapeDtypeStruct((B, EMB_DIM), jnp.float32),
        mesh=mesh,
        scratch_types=[
            pltpu.VMEM((b_per_w,), jnp.int32),
            pltpu.VMEM((CHUNK, EMB_DIM), jnp.float32),
            pltpu.VMEM((CHUNK, EMB_DIM), jnp.float32),
            pltpu.SemaphoreType.DMA,
            pltpu.SemaphoreType.DMA,
            pltpu.SemaphoreType.DMA,
            pltpu.SemaphoreType.DMA,
        ],
        compiler_params=pltpu.CompilerParams(use_tc_tiling_on_sc=False),
    )
    def k(idx_hbm, table_hbm, out_hbm, idx_v, rows0, rows1, g0, g1, o0, o1):
        wid = lax.axis_index("s") * NC + lax.axis_index("c")
        base = wid * b_per_w
        pltpu.sync_copy(idx_hbm.at[pl.ds(base, b_per_w)], idx_v)

        bufs = [(rows0, g0, o0), (rows1, g1, o1)]

        def gather(c):
            rows, gsem, _ = bufs[c % 2]
            return pltpu.async_copy(
                table_hbm.at[idx_v.at[pl.ds(c * CHUNK, CHUNK)]], rows, gsem
            )

        def writeback(c):
            rows, _, osem = bufs[c % 2]
            return pltpu.async_copy(
                rows, out_hbm.at[pl.ds(base + c * CHUNK, CHUNK)], osem
            )

        g_desc = {0: gather(0)}
        o_desc = {}
        for c in range(n_chunks):
            if c + 1 < n_chunks:
                if c >= 1:
                    o_desc[c - 1].wait()
                g_desc[c + 1] = gather(c + 1)
            g_desc[c].wait()
            o_desc[c] = writeback(c)
        o_desc[n_chunks - 2].wait()
        o_desc[n_chunks - 1].wait()

    return k(idx, table)


def kernel(cond_ids, emb_weight):
    batch, nf = cond_ids.shape
    B = batch * nf
    idx = cond_ids.astype(jnp.int32).reshape(B)
    out = _gather_rows(B, idx, emb_weight)
    return out.reshape(batch, nf, EMB_DIM)


# speedup vs baseline: 1.6717x; 1.0602x over previous
import functools

import jax
import jax.numpy as jnp
from jax import lax
from jax.experimental import pallas as pl
from jax.experimental.pallas import tpu as pltpu
from jax.experimental.pallas import tpu_sc as plsc

EMB_DIM = 32
BATCH = 16384
NF = 26

NC = 2
NS = 16
NW = NC * NS

CHUNK = 1024
NBLK = BATCH // CHUNK
N_UNITS = NF * NBLK
UPW = N_UNITS // NW


@jax.jit
def _gather_rows(idx, table):
    mesh = plsc.VectorSubcoreMesh(core_axis_name="c", subcore_axis_name="s")

    @functools.partial(
        pl.kernel,
        out_type=jax.ShapeDtypeStruct((NF, BATCH, EMB_DIM), jnp.float32),
        mesh=mesh,
        scratch_types=[
            pltpu.VMEM((UPW, CHUNK), jnp.int32),
            pltpu.VMEM((CHUNK, EMB_DIM), jnp.float32),
            pltpu.VMEM((CHUNK, EMB_DIM), jnp.float32),
            pltpu.SemaphoreType.DMA,
            pltpu.SemaphoreType.DMA,
            pltpu.SemaphoreType.DMA,
            pltpu.SemaphoreType.DMA,
        ],
        compiler_params=pltpu.CompilerParams(use_tc_tiling_on_sc=False),
    )
    def k(idx_hbm, table_hbm, out_hbm, idx_v, rows0, rows1, g0, g1, o0, o1):
        wid = lax.axis_index("s") * NC + lax.axis_index("c")
        u0 = wid * UPW
        pltpu.sync_copy(idx_hbm.at[pl.ds(u0, UPW)], idx_v)

        bufs = [(rows0, g0, o0), (rows1, g1, o1)]

        def gather(j):
            rows, gsem, _ = bufs[j % 2]
            return pltpu.async_copy(
                table_hbm.at[idx_v.at[j]], rows, gsem
            )

        def writeback(j):
            rows, _, osem = bufs[j % 2]
            u = u0 + j
            f = u // NBLK
            b0 = (u % NBLK) * CHUNK
            return pltpu.async_copy(
                rows, out_hbm.at[f, pl.ds(b0, CHUNK)], osem
            )

        g_desc = {0: gather(0)}
        o_desc = {}
        for j in range(UPW):
            if j + 1 < UPW:
                if j >= 1:
                    o_desc[j - 1].wait()
                g_desc[j + 1] = gather(j + 1)
            g_desc[j].wait()
            o_desc[j] = writeback(j)
        o_desc[UPW - 2].wait()
        o_desc[UPW - 1].wait()

    return k(idx, table)


def kernel(cond_ids, emb_weight):
    idx = cond_ids.astype(jnp.int32).T.reshape(N_UNITS, CHUNK)
    out = _gather_rows(idx, emb_weight)
    return out.transpose(1, 0, 2)


# speedup vs baseline: 1.6735x; 1.0011x over previous
import functools

import jax
import jax.numpy as jnp
from jax import lax
from jax.experimental import pallas as pl
from jax.experimental.pallas import tpu as pltpu
from jax.experimental.pallas import tpu_sc as plsc

EMB_DIM = 32
BATCH = 16384
NF = 26

NC = 2
NS = 16
NW = NC * NS

CHUNK = 1024
NBLK = BATCH // CHUNK
N_UNITS = NF * NBLK
UPW = N_UNITS // NW


@jax.jit
def _gather_rows(idx, table):
    mesh = plsc.VectorSubcoreMesh(core_axis_name="c", subcore_axis_name="s")

    @functools.partial(
        pl.kernel,
        out_type=jax.ShapeDtypeStruct((NF, BATCH, EMB_DIM), jnp.float32),
        mesh=mesh,
        scratch_types=[
            pltpu.VMEM((UPW * CHUNK,), jnp.int32),
            pltpu.VMEM((CHUNK, EMB_DIM), jnp.float32),
            pltpu.VMEM((CHUNK, EMB_DIM), jnp.float32),
            pltpu.SemaphoreType.DMA,
            pltpu.SemaphoreType.DMA,
            pltpu.SemaphoreType.DMA,
            pltpu.SemaphoreType.DMA,
        ],
        compiler_params=pltpu.CompilerParams(use_tc_tiling_on_sc=False),
    )
    def k(idx_hbm, table_hbm, out_hbm, idx_v, rows0, rows1, g0, g1, o0, o1):
        wid = lax.axis_index("s") * NC + lax.axis_index("c")
        u0 = wid * UPW
        pltpu.sync_copy(idx_hbm.at[pl.ds(u0 * CHUNK, UPW * CHUNK)], idx_v)

        bufs = [(rows0, g0, o0), (rows1, g1, o1)]

        def gather(j):
            rows, gsem, _ = bufs[j % 2]
            return pltpu.async_copy(
                table_hbm.at[idx_v.at[pl.ds(j * CHUNK, CHUNK)]], rows, gsem
            )

        def writeback(j):
            rows, _, osem = bufs[j % 2]
            u = u0 + j
            f = u // NBLK
            b0 = (u % NBLK) * CHUNK
            return pltpu.async_copy(
                rows, out_hbm.at[f, pl.ds(b0, CHUNK)], osem
            )

        g_desc = {0: gather(0)}
        o_desc = {}
        for j in range(UPW):
            if j + 1 < UPW:
                if j >= 1:
                    o_desc[j - 1].wait()
                g_desc[j + 1] = gather(j + 1)
            g_desc[j].wait()
            o_desc[j] = writeback(j)
        o_desc[UPW - 2].wait()
        o_desc[UPW - 1].wait()

    return k(idx, table)


def kernel(cond_ids, emb_weight):
    idx = cond_ids.astype(jnp.int32).T.reshape(N_UNITS * CHUNK)
    out = _gather_rows(idx, emb_weight)
    return out.transpose(1, 0, 2)
